# Initial kernel scaffold; baseline (speedup 1.0000x reference)
import jax
import jax.numpy as jnp
from jax.experimental import pallas as pl


def kernel(x, edge_index, W1l, W1r, b1, W2l, W2r, b2, W3l, W3r, b3):
    raise NotImplementedError("write your pallas kernel here")


# speedup vs baseline: 7.2146x; 7.2146x over previous
import functools

import jax
import jax.numpy as jnp
from jax import lax
from jax.experimental import pallas as pl
from jax.experimental.pallas import tpu as pltpu
from jax.experimental.pallas import tpu_sc as plsc

_NCORES = 2
_NSUB = 16
_NW = _NCORES * _NSUB
_C = 80


def _make_agg(n, npad, d, nchunk, compute_deg):
    mesh = plsc.VectorSubcoreMesh(core_axis_name="c", subcore_axis_name="s")
    rows_per_tile = npad // _NSUB

    out_type = [jax.ShapeDtypeStruct((_NCORES, npad, d), jnp.float32)]
    scratch = [
        pltpu.VMEM((nchunk, _C), jnp.int32),
        pltpu.VMEM((nchunk, _C), jnp.int32),
        pltpu.VMEM((_C, d), jnp.float32),
        pltpu.VMEM((16, d), jnp.float32),
        pltpu.VMEM_SHARED((npad, d), jnp.float32),
    ]
    if compute_deg:
        out_type.append(jax.ShapeDtypeStruct((_NCORES, npad), jnp.float32))
        scratch.append(pltpu.VMEM((_C,), jnp.float32))
        scratch.append(pltpu.VMEM_SHARED((npad,), jnp.float32))

    def body(h_hbm, src_hbm, dst_hbm, acc_hbm, *rest):
        if compute_deg:
            deg_hbm, src_v, dst_v, rows_v, zero_v, acc_sh, ones_v, deg_sh = rest
        else:
            src_v, dst_v, rows_v, zero_v, acc_sh = rest
        cid = lax.axis_index("c")
        tid = lax.axis_index("s")
        w = cid * _NSUB + tid
        base = tid * rows_per_tile

        @pl.loop(0, 16)
        def _(r):
            @pl.loop(0, d // 16)
            def _(k):
                zero_v[r, pl.ds(k * 16, 16)] = jnp.zeros((16,), jnp.float32)

        if compute_deg:
            @pl.loop(0, _C // 16)
            def _(k):
                ones_v[pl.ds(k * 16, 16)] = jnp.ones((16,), jnp.float32)

        @pl.loop(0, rows_per_tile // 16)
        def _(r):
            pltpu.sync_copy(zero_v, acc_sh.at[pl.ds(base + r * 16, 16)])
        if compute_deg:
            @pl.loop(0, rows_per_tile // d)
            def _(r):
                pltpu.sync_copy(zero_v.at[0], deg_sh.at[pl.ds(base + r * d, d)])
        plsc.subcore_barrier()

        pltpu.sync_copy(src_hbm.at[w], src_v)
        pltpu.sync_copy(dst_hbm.at[w], dst_v)

        @pl.loop(0, nchunk)
        def _(j):
            pltpu.sync_copy(h_hbm.at[src_v.at[j]], rows_v)
            pltpu.sync_copy(rows_v, acc_sh.at[dst_v.at[j]], add=True)
            if compute_deg:
                pltpu.sync_copy(ones_v, deg_sh.at[dst_v.at[j]], add=True)

        plsc.subcore_barrier()
        pltpu.sync_copy(acc_sh.at[pl.ds(base, rows_per_tile)],
                        acc_hbm.at[cid, pl.ds(base, rows_per_tile)])
        if compute_deg:
            pltpu.sync_copy(deg_sh.at[pl.ds(base, rows_per_tile)],
                            deg_hbm.at[cid, pl.ds(base, rows_per_tile)])

    return pl.kernel(body, out_type=tuple(out_type), mesh=mesh,
                     scratch_types=scratch)


def _layer_body(relu, a0_ref, a1_ref, d0_ref, d1_ref, h_ref, wl_ref, wr_ref,
                b_ref, out_ref):
    agg = a0_ref[0] + a1_ref[0]
    deg = jnp.maximum(d0_ref[0] + d1_ref[0], 1.0)
    dn = (((1,), (1,)), ((), ()))
    z = lax.dot_general(agg / deg, wl_ref[...], dn,
                        precision=lax.Precision.HIGHEST)
    z = z + lax.dot_general(h_ref[...], wr_ref[...], dn,
                            precision=lax.Precision.HIGHEST)
    z = z + b_ref[...]
    out_ref[...] = jnp.maximum(z, 0.0) if relu else z


def _layer(accP, degP, h, Wl, Wr, b, relu):
    n, d = h.shape
    r = 400
    return pl.pallas_call(
        functools.partial(_layer_body, relu),
        grid=(n // r,),
        in_specs=[
            pl.BlockSpec((1, r, d), lambda i: (0, i, 0)),
            pl.BlockSpec((1, r, d), lambda i: (1, i, 0)),
            pl.BlockSpec((1, r, 1), lambda i: (0, i, 0)),
            pl.BlockSpec((1, r, 1), lambda i: (1, i, 0)),
            pl.BlockSpec((r, d), lambda i: (i, 0)),
            pl.BlockSpec((d, d), lambda i: (0, 0)),
            pl.BlockSpec((d, d), lambda i: (0, 0)),
            pl.BlockSpec((1, d), lambda i: (0, 0)),
        ],
        out_specs=pl.BlockSpec((r, d), lambda i: (i, 0)),
        out_shape=jax.ShapeDtypeStruct((n, d), jnp.float32),
    )(accP, accP, degP, degP, h, Wl, Wr, b)


def kernel(x, edge_index, W1l, W1r, b1, W2l, W2r, b2, W3l, W3r, b3):
    n, d = x.shape
    e = edge_index.shape[1]
    per_w = e // _NW
    nchunk = per_w // _C
    npad = ((n + 2047) // 2048) * 2048

    src = edge_index[0].reshape(_NW, nchunk, _C)
    dst = edge_index[1].reshape(_NW, nchunk, _C)

    agg_deg = _make_agg(n, npad, d, nchunk, True)
    agg = _make_agg(n, npad, d, nchunk, False)

    accP, degP = agg_deg(x, src, dst)
    degP = degP.reshape(_NCORES, npad, 1)
    h = _layer(accP, degP, x, W1l, W1r, b1.reshape(1, d), True)
    accP = agg(h, src, dst)
    accP = accP[0] if isinstance(accP, (tuple, list)) else accP
    h = _layer(accP, degP, h, W2l, W2r, b2.reshape(1, d), True)
    accP = agg(h, src, dst)
    accP = accP[0] if isinstance(accP, (tuple, list)) else accP
    return _layer(accP, degP, h, W3l, W3r, b3.reshape(1, d), False)
